# baseline (speedup 1.0000x reference)
import functools

import jax
import jax.numpy as jnp
from jax import lax
from jax.experimental import pallas as pl
from jax.experimental.pallas import tpu as pltpu
from jax.experimental.pallas import tpu_sc as plsc

_B = 16384
_EMB = 128
_LAT = 64

_CHUNK = 256
_NBUF = 3
_WDELAY = 2


@functools.lru_cache(maxsize=None)
def _make_gather3():
    info = plsc.get_sparse_core_info()
    nc, ns = info.num_cores, info.num_subcores
    nw = nc * ns
    bpw = _B // nw
    nch = bpw // _CHUNK
    nunits = 3 * nch
    mesh = plsc.VectorSubcoreMesh(core_axis_name="c", subcore_axis_name="s")

    @functools.partial(
        pl.kernel,
        out_type=(jax.ShapeDtypeStruct((_B, _EMB), jnp.float32),) * 3,
        mesh=mesh,
        scratch_types=[pltpu.VMEM((_CHUNK,), jnp.int32)] * nunits
        + [pltpu.VMEM((_CHUNK, _EMB), jnp.float32)] * _NBUF
        + [pltpu.SemaphoreType.DMA] * (2 * _NBUF),
    )
    def _gather3(g_hbm, b_hbm, p_hbm, gt_hbm, bt_hbm, pt_hbm,
                 og_hbm, ob_hbm, op_hbm, *scr):
        idx_v = scr[:nunits]
        bufs = scr[nunits:nunits + _NBUF]
        gsems = scr[nunits + _NBUF:nunits + 2 * _NBUF]
        wsems = scr[nunits + 2 * _NBUF:]
        wid = lax.axis_index("s") * nc + lax.axis_index("c")
        base = wid * bpw
        idxs = (g_hbm, b_hbm, p_hbm)
        tbls = (gt_hbm, bt_hbm, pt_hbm)
        outs = (og_hbm, ob_hbm, op_hbm)
        units = [(t, c) for t in range(3) for c in range(nch)]
        icp = [pltpu.async_copy(idxs[t].at[pl.ds(base + c * _CHUNK, _CHUNK)],
                                idx_v[u], wsems[u % _NBUF])
               for u, (t, c) in enumerate(units)]
        for cp in icp:
            cp.wait()

        gcp = [None] * _NBUF
        wcp = [None] * _NBUF

        def write_unit(v):
            vb = v % _NBUF
            vt, vc = units[v]
            gcp[vb].wait()
            wcp[vb] = pltpu.async_copy(
                bufs[vb], outs[vt].at[pl.ds(base + vc * _CHUNK, _CHUNK)],
                wsems[vb])

        for u, (t, c) in enumerate(units):
            b = u % _NBUF
            if u >= _NBUF:
                wcp[b].wait()
            gcp[b] = pltpu.async_copy(tbls[t].at[idx_v[u]], bufs[b],
                                      gsems[b])
            if u >= _WDELAY:
                write_unit(u - _WDELAY)
        for v in range(nunits - _WDELAY, nunits):
            write_unit(v)
        for v in range(max(0, nunits - _NBUF), nunits):
            wcp[v % _NBUF].wait()

    return _gather3


_BS = 16384


def _proj_body(g_ref, b_ref, p_ref, wp_ref, bp_ref, wf_ref, bf_ref, o_ref):
    c = jnp.dot(g_ref[...], wp_ref[0:_EMB, :], preferred_element_type=jnp.float32)
    c += jnp.dot(b_ref[...], wp_ref[_EMB:2 * _EMB, :], preferred_element_type=jnp.float32)
    c += jnp.dot(p_ref[...], wp_ref[2 * _EMB:3 * _EMB, :], preferred_element_type=jnp.float32)
    c += bp_ref[...]
    o_ref[...] = jnp.dot(c, wf_ref[...], preferred_element_type=jnp.float32) + bf_ref[...]


def _project(g_embs, b_embs, p_embs, W_proj, b_proj, W_fc, b_fc):
    grid = (_B // _BS,)
    row_spec = pl.BlockSpec((_BS, _EMB), lambda i: (i, 0))
    full = lambda shape: pl.BlockSpec(shape, lambda i: (0,) * len(shape))
    return pl.pallas_call(
        _proj_body,
        grid=grid,
        in_specs=[
            row_spec, row_spec, row_spec,
            full((3 * _EMB, _EMB)),
            full((1, _EMB)),
            full((_EMB, _LAT)),
            full((1, _LAT)),
        ],
        out_specs=pl.BlockSpec((_BS, _LAT), lambda i: (i, 0)),
        out_shape=jax.ShapeDtypeStruct((_B, _LAT), jnp.float32),
    )(g_embs, b_embs, p_embs, W_proj, b_proj.reshape(1, _EMB),
      W_fc, b_fc.reshape(1, _LAT))


def kernel(gains, bws, pms, gain_table, bw_table, pm_table,
           W_proj, b_proj, W_fc, b_fc):
    gains = gains.astype(jnp.int32)
    bws = bws.astype(jnp.int32)
    pms = pms.astype(jnp.int32)
    g_embs, b_embs, p_embs = _make_gather3()(
        gains, bws, pms, gain_table, bw_table, pm_table)
    return _project(g_embs, b_embs, p_embs, W_proj, b_proj, W_fc, b_fc)


# speedup vs baseline: 1.0679x; 1.0679x over previous
import functools

import jax
import jax.numpy as jnp
from jax import lax
from jax.experimental import pallas as pl
from jax.experimental.pallas import tpu as pltpu
from jax.experimental.pallas import tpu_sc as plsc

_B = 16384
_EMB = 128
_LAT = 64

_CHUNK = 256
_NBUF = 3
_WDELAY = 2


@functools.lru_cache(maxsize=None)
def _make_gather3():
    info = plsc.get_sparse_core_info()
    nc, ns = info.num_cores, info.num_subcores
    nw = nc * ns
    bpw = _B // nw
    nch = bpw // _CHUNK
    nunits = 3 * nch
    mesh = plsc.VectorSubcoreMesh(core_axis_name="c", subcore_axis_name="s")

    @functools.partial(
        pl.kernel,
        out_type=(jax.ShapeDtypeStruct((_B, _EMB), jnp.float32),) * 3,
        mesh=mesh,
        scratch_types=[pltpu.VMEM((_CHUNK,), jnp.int32)] * nunits
        + [pltpu.VMEM((_CHUNK, _EMB), jnp.float32)] * _NBUF
        + [pltpu.SemaphoreType.DMA] * (2 * _NBUF),
    )
    def _gather3(g_hbm, b_hbm, p_hbm, gt_hbm, bt_hbm, pt_hbm,
                 og_hbm, ob_hbm, op_hbm, *scr):
        idx_v = scr[:nunits]
        bufs = scr[nunits:nunits + _NBUF]
        gsems = scr[nunits + _NBUF:nunits + 2 * _NBUF]
        wsems = scr[nunits + 2 * _NBUF:]
        wid = lax.axis_index("s") * nc + lax.axis_index("c")
        base = wid * bpw
        idxs = (g_hbm, b_hbm, p_hbm)
        tbls = (gt_hbm, bt_hbm, pt_hbm)
        outs = (og_hbm, ob_hbm, op_hbm)
        units = [(t, c) for t in range(3) for c in range(nch)]
        icp = [pltpu.async_copy(idxs[t].at[pl.ds(base + c * _CHUNK, _CHUNK)],
                                idx_v[u], wsems[u % _NBUF])
               for u, (t, c) in enumerate(units)]
        for cp in icp:
            cp.wait()

        gcp = [None] * _NBUF
        wcp = [None] * _NBUF

        def write_unit(v):
            vb = v % _NBUF
            vt, vc = units[v]
            gcp[vb].wait()
            wcp[vb] = pltpu.async_copy(
                bufs[vb], outs[vt].at[pl.ds(base + vc * _CHUNK, _CHUNK)],
                wsems[vb])

        for u, (t, c) in enumerate(units):
            b = u % _NBUF
            if u >= _NBUF:
                wcp[b].wait()
            gcp[b] = pltpu.async_copy(tbls[t].at[idx_v[u]], bufs[b],
                                      gsems[b])
            if u >= _WDELAY:
                write_unit(u - _WDELAY)
        for v in range(nunits - _WDELAY, nunits):
            write_unit(v)
        for v in range(max(0, nunits - _NBUF), nunits):
            wcp[v % _NBUF].wait()

    return _gather3


_BS = 8192


def _proj_body(g_ref, b_ref, p_ref, wp_ref, bp_ref, wf_ref, bf_ref, o_ref):
    c = jnp.dot(g_ref[...], wp_ref[0:_EMB, :], preferred_element_type=jnp.float32)
    c += jnp.dot(b_ref[...], wp_ref[_EMB:2 * _EMB, :], preferred_element_type=jnp.float32)
    c += jnp.dot(p_ref[...], wp_ref[2 * _EMB:3 * _EMB, :], preferred_element_type=jnp.float32)
    c += bp_ref[...]
    o_ref[...] = jnp.dot(c, wf_ref[...], preferred_element_type=jnp.float32) + bf_ref[...]


def _project(g_embs, b_embs, p_embs, W_proj, b_proj, W_fc, b_fc):
    grid = (_B // _BS,)
    row_spec = pl.BlockSpec((_BS, _EMB), lambda i: (i, 0))
    full = lambda shape: pl.BlockSpec(shape, lambda i: (0,) * len(shape))
    return pl.pallas_call(
        _proj_body,
        grid=grid,
        in_specs=[
            row_spec, row_spec, row_spec,
            full((3 * _EMB, _EMB)),
            full((1, _EMB)),
            full((_EMB, _LAT)),
            full((1, _LAT)),
        ],
        out_specs=pl.BlockSpec((_BS, _LAT), lambda i: (i, 0)),
        out_shape=jax.ShapeDtypeStruct((_B, _LAT), jnp.float32),
    )(g_embs, b_embs, p_embs, W_proj, b_proj.reshape(1, _EMB),
      W_fc, b_fc.reshape(1, _LAT))


def kernel(gains, bws, pms, gain_table, bw_table, pm_table,
           W_proj, b_proj, W_fc, b_fc):
    gains = gains.astype(jnp.int32)
    bws = bws.astype(jnp.int32)
    pms = pms.astype(jnp.int32)
    g_embs, b_embs, p_embs = _make_gather3()(
        gains, bws, pms, gain_table, bw_table, pm_table)
    return _project(g_embs, b_embs, p_embs, W_proj, b_proj, W_fc, b_fc)
